# baseline (speedup 1.0000x reference)
import functools

import jax
import jax.numpy as jnp
from jax import lax
from jax.experimental import pallas as pl
from jax.experimental.pallas import tpu as pltpu
from jax.experimental.pallas import tpu_sc as plsc

VOCAB = 100000
D = 128
B = 4096
NCTX = 20
NPAIR = B * NCTX

NC = 2
NS = 16
L = 16
NW = NC * NS

CPW = B // NW
CC = 8
NT = CPW // CC
PC = CC * NCTX
IW = 80
NIR = PC // IW
NQ = D // L


def _rsqrt(t):
    ti = plsc.bitcast(t, jnp.int32)
    y = plsc.bitcast(jnp.int32(0x5F3759DF) - (ti >> 1), jnp.float32)
    for _ in range(3):
        y = y * (1.5 - 0.5 * t * y * y)
    return y


def _sc_body(centers_hbm, contexts_hbm, cidx_hbm, ctxidx_hbm, negidx_hbm,
             out_hbm,
             cidx_v, ctxidx_v, negidx_v, c_rows, ctx_rows, neg_rows,
             n2c_buf, dp_buf, n2x_buf, dn_buf, n2n_buf, kidx_v, acc_v,
             *sems):
    w = lax.axis_index("s") * NC + lax.axis_index("c")
    last_lane = lax.iota(jnp.int32, L) == (L - 1)
    lane = lax.iota(jnp.int32, L)
    for g in range(PC // L):
        kidx_v[pl.ds(g * L, L)] = (g * L + lane) // NCTX

    def load_indices():
        cbase = pl.multiple_of(w * NT, NT)
        pltpu.sync_copy(cidx_hbm.at[pl.ds(cbase, NT)], cidx_v)
        rbase = pl.multiple_of(w * (NT * NIR), NT * NIR)
        pltpu.sync_copy(ctxidx_hbm.at[pl.ds(rbase, NT * NIR)], ctxidx_v)
        pltpu.sync_copy(negidx_hbm.at[pl.ds(rbase, NT * NIR)], negidx_v)

    def stage(t, s):
        cs = [pltpu.async_copy(
            centers_hbm.at[cidx_v.at[t]], c_rows.at[s], sems[s])]
        for j in range(NIR):
            cs.append(pltpu.async_copy(
                contexts_hbm.at[ctxidx_v.at[t * NIR + j]],
                ctx_rows.at[s].at[pl.ds(j * IW, IW)], sems[s]))
            cs.append(pltpu.async_copy(
                contexts_hbm.at[negidx_v.at[t * NIR + j]],
                neg_rows.at[s].at[pl.ds(j * IW, IW)], sems[s]))
        return cs

    def store_total(buf, p, vec):
        plsc.store_scatter(buf, [jnp.full((L,), p, jnp.int32)],
                           plsc.cumsum(vec), mask=last_lane)

    def compute(s, acc):
        def center_body(k, _):
            cqs = [c_rows[s, k, pl.ds(q * L, L)] for q in range(NQ)]
            n2cv = cqs[0] * cqs[0]
            for q in range(1, NQ):
                n2cv = n2cv + cqs[q] * cqs[q]
            store_total(n2c_buf, k, n2cv)

            @plsc.parallel_loop(0, NCTX, unroll=4)
            def pair_body(j):
                p = k * NCTX + j
                xv = ctx_rows[s, p, pl.ds(0, L)]
                nv = neg_rows[s, p, pl.ds(0, L)]
                dpv = xv * cqs[0]
                n2xv = xv * xv
                dnv = nv * cqs[0]
                n2nv = nv * nv
                for q in range(1, NQ):
                    xv = ctx_rows[s, p, pl.ds(q * L, L)]
                    nv = neg_rows[s, p, pl.ds(q * L, L)]
                    dpv = dpv + xv * cqs[q]
                    n2xv = n2xv + xv * xv
                    dnv = dnv + nv * cqs[q]
                    n2nv = n2nv + nv * nv
                store_total(dp_buf, p, dpv)
                store_total(n2x_buf, p, n2xv)
                store_total(dn_buf, p, dnv)
                store_total(n2n_buf, p, n2nv)

            return 0

        lax.fori_loop(0, CC, center_body, 0)

        @plsc.parallel_loop(0, PC // L, unroll=2, carry=acc)
        def grp_body(g, a):
            off = g * L
            dpv = dp_buf[pl.ds(off, L)]
            n2xv = n2x_buf[pl.ds(off, L)]
            dnv = dn_buf[pl.ds(off, L)]
            n2nv = n2n_buf[pl.ds(off, L)]
            kidx = kidx_v[pl.ds(off, L)]
            n2cv = plsc.load_gather(n2c_buf, [kidx])
            tp = n2cv * n2xv
            denp = jnp.maximum(tp * _rsqrt(tp), 1e-8)
            cosp = dpv / denp
            tn = n2cv * n2nv
            denn = jnp.maximum(tn * _rsqrt(tn), 1e-8)
            cosn = dnv / denn
            return a + (1.0 - cosp) + jnp.maximum(cosn, 0.0)

        return grp_body

    acc = jnp.zeros((L,), jnp.float32)
    load_indices()
    descs = {0: stage(0, 0)}
    for t in range(NT):
        s = t % 2
        if t + 1 < NT:
            descs[1 - s] = stage(t + 1, 1 - s)
        for c in descs[s]:
            c.wait()
        acc = compute(s, acc)

    acc_v[...] = acc * (1.0 / NPAIR)
    pltpu.sync_copy(acc_v, out_hbm.at[w])


_sc_kernel = functools.partial(
    pl.kernel,
    out_type=jax.ShapeDtypeStruct((NW, L), jnp.float32),
    mesh=plsc.VectorSubcoreMesh(core_axis_name="c", subcore_axis_name="s"),
    compiler_params=pltpu.CompilerParams(needs_layout_passes=False),
    scratch_types=[
        pltpu.VMEM((NT, CC), jnp.int32),
        pltpu.VMEM((NT * NIR, IW), jnp.int32),
        pltpu.VMEM((NT * NIR, IW), jnp.int32),
        pltpu.VMEM((2, CC, D), jnp.float32),
        pltpu.VMEM((2, PC, D), jnp.float32),
        pltpu.VMEM((2, PC, D), jnp.float32),
        pltpu.VMEM((CC,), jnp.float32),
        pltpu.VMEM((PC,), jnp.float32),
        pltpu.VMEM((PC,), jnp.float32),
        pltpu.VMEM((PC,), jnp.float32),
        pltpu.VMEM((PC,), jnp.float32),
        pltpu.VMEM((PC,), jnp.int32),
        pltpu.VMEM((L,), jnp.float32),
        pltpu.SemaphoreType.DMA,
        pltpu.SemaphoreType.DMA,
    ],
)(_sc_body)


@jax.jit
def kernel(centers, contexts, center_idxs, context_idxs, neg_idxs):
    cidx = center_idxs.astype(jnp.int32).reshape(B // CC, CC)
    ctxi = context_idxs.astype(jnp.int32).reshape(NPAIR // IW, IW)
    negi = neg_idxs.astype(jnp.int32).reshape(NPAIR // IW, IW)
    out = _sc_kernel(centers, contexts, cidx, ctxi, negi)
    return jnp.sum(out)


# speedup vs baseline: 1.0485x; 1.0485x over previous
import functools

import jax
import jax.numpy as jnp
from jax import lax
from jax.experimental import pallas as pl
from jax.experimental.pallas import tpu as pltpu
from jax.experimental.pallas import tpu_sc as plsc

VOCAB = 100000
D = 128
B = 4096
NCTX = 20
NPAIR = B * NCTX

NC = 2
NS = 16
L = 16
NW = NC * NS

CPW = B // NW
CC = 8
NT = CPW // CC
PC = CC * NCTX
IW = 80
NIR = PC // IW
NQ = D // L


def _rsqrt(t):
    ti = plsc.bitcast(t, jnp.int32)
    y = plsc.bitcast(jnp.int32(0x5F3759DF) - (ti >> 1), jnp.float32)
    for _ in range(3):
        y = y * (1.5 - 0.5 * t * y * y)
    return y


def _sc_body(centers_hbm, contexts_hbm, cidx_hbm, ctxidx_hbm, negidx_hbm,
             out_hbm,
             cidx_v, ctxidx_v, negidx_v, c_rows, ctx_rows, neg_rows,
             n2c_buf, dp_buf, n2x_buf, dn_buf, n2n_buf, kidx_v, acc_v,
             *sems):
    w = lax.axis_index("s") * NC + lax.axis_index("c")
    last_lane = lax.iota(jnp.int32, L) == (L - 1)
    lane = lax.iota(jnp.int32, L)
    for g in range(PC // L):
        kidx_v[pl.ds(g * L, L)] = (g * L + lane) // NCTX

    def load_indices():
        cbase = pl.multiple_of(w * NT, NT)
        pltpu.sync_copy(cidx_hbm.at[pl.ds(cbase, NT)], cidx_v)
        rbase = pl.multiple_of(w * (NT * NIR), NT * NIR)
        pltpu.sync_copy(ctxidx_hbm.at[pl.ds(rbase, NT * NIR)], ctxidx_v)
        pltpu.sync_copy(negidx_hbm.at[pl.ds(rbase, NT * NIR)], negidx_v)

    def stage(t, s):
        cs = [pltpu.async_copy(
            centers_hbm.at[cidx_v.at[t]], c_rows.at[s], sems[s])]
        for j in range(NIR):
            cs.append(pltpu.async_copy(
                contexts_hbm.at[ctxidx_v.at[t * NIR + j]],
                ctx_rows.at[s].at[pl.ds(j * IW, IW)], sems[s]))
            cs.append(pltpu.async_copy(
                contexts_hbm.at[negidx_v.at[t * NIR + j]],
                neg_rows.at[s].at[pl.ds(j * IW, IW)], sems[s]))
        return cs

    def store_total(buf, p, vec):
        plsc.store_scatter(buf, [jnp.full((L,), p, jnp.int32)],
                           plsc.cumsum(vec), mask=last_lane)

    def compute(s, acc):
        def center_body(k, _):
            cqs = [c_rows[s, k, pl.ds(q * L, L)] for q in range(NQ)]
            n2cv = cqs[0] * cqs[0]
            for q in range(1, NQ):
                n2cv = n2cv + cqs[q] * cqs[q]
            store_total(n2c_buf, k, n2cv)

            @plsc.parallel_loop(0, NCTX, unroll=2)
            def pair_body(j):
                p = k * NCTX + j
                xv = ctx_rows[s, p, pl.ds(0, L)]
                nv = neg_rows[s, p, pl.ds(0, L)]
                dpv = xv * cqs[0]
                n2xv = xv * xv
                dnv = nv * cqs[0]
                n2nv = nv * nv
                for q in range(1, NQ):
                    xv = ctx_rows[s, p, pl.ds(q * L, L)]
                    nv = neg_rows[s, p, pl.ds(q * L, L)]
                    dpv = dpv + xv * cqs[q]
                    n2xv = n2xv + xv * xv
                    dnv = dnv + nv * cqs[q]
                    n2nv = n2nv + nv * nv
                store_total(dp_buf, p, dpv)
                store_total(n2x_buf, p, n2xv)
                store_total(dn_buf, p, dnv)
                store_total(n2n_buf, p, n2nv)

            return 0

        lax.fori_loop(0, CC, center_body, 0)

        @plsc.parallel_loop(0, PC // L, unroll=2, carry=acc)
        def grp_body(g, a):
            off = g * L
            dpv = dp_buf[pl.ds(off, L)]
            n2xv = n2x_buf[pl.ds(off, L)]
            dnv = dn_buf[pl.ds(off, L)]
            n2nv = n2n_buf[pl.ds(off, L)]
            kidx = kidx_v[pl.ds(off, L)]
            n2cv = plsc.load_gather(n2c_buf, [kidx])
            tp = n2cv * n2xv
            denp = jnp.maximum(tp * _rsqrt(tp), 1e-8)
            cosp = dpv / denp
            tn = n2cv * n2nv
            denn = jnp.maximum(tn * _rsqrt(tn), 1e-8)
            cosn = dnv / denn
            return a + (1.0 - cosp) + jnp.maximum(cosn, 0.0)

        return grp_body

    acc = jnp.zeros((L,), jnp.float32)
    load_indices()
    descs = {0: stage(0, 0)}
    for t in range(NT):
        s = t % 2
        if t + 1 < NT:
            descs[1 - s] = stage(t + 1, 1 - s)
        for c in descs[s]:
            c.wait()
        acc = compute(s, acc)

    acc_v[...] = acc * (1.0 / NPAIR)
    pltpu.sync_copy(acc_v, out_hbm.at[w])


_sc_kernel = functools.partial(
    pl.kernel,
    out_type=jax.ShapeDtypeStruct((NW, L), jnp.float32),
    mesh=plsc.VectorSubcoreMesh(core_axis_name="c", subcore_axis_name="s"),
    compiler_params=pltpu.CompilerParams(needs_layout_passes=False),
    scratch_types=[
        pltpu.VMEM((NT, CC), jnp.int32),
        pltpu.VMEM((NT * NIR, IW), jnp.int32),
        pltpu.VMEM((NT * NIR, IW), jnp.int32),
        pltpu.VMEM((2, CC, D), jnp.float32),
        pltpu.VMEM((2, PC, D), jnp.float32),
        pltpu.VMEM((2, PC, D), jnp.float32),
        pltpu.VMEM((CC,), jnp.float32),
        pltpu.VMEM((PC,), jnp.float32),
        pltpu.VMEM((PC,), jnp.float32),
        pltpu.VMEM((PC,), jnp.float32),
        pltpu.VMEM((PC,), jnp.float32),
        pltpu.VMEM((PC,), jnp.int32),
        pltpu.VMEM((L,), jnp.float32),
        pltpu.SemaphoreType.DMA,
        pltpu.SemaphoreType.DMA,
    ],
)(_sc_body)


@jax.jit
def kernel(centers, contexts, center_idxs, context_idxs, neg_idxs):
    cidx = center_idxs.astype(jnp.int32).reshape(B // CC, CC)
    ctxi = context_idxs.astype(jnp.int32).reshape(NPAIR // IW, IW)
    negi = neg_idxs.astype(jnp.int32).reshape(NPAIR // IW, IW)
    out = _sc_kernel(centers, contexts, cidx, ctxi, negi)
    return jnp.sum(out)


# speedup vs baseline: 1.0742x; 1.0246x over previous
import functools

import jax
import jax.numpy as jnp
from jax import lax
from jax.experimental import pallas as pl
from jax.experimental.pallas import tpu as pltpu
from jax.experimental.pallas import tpu_sc as plsc

VOCAB = 100000
D = 128
B = 4096
NCTX = 20
NPAIR = B * NCTX

NC = 2
NS = 16
L = 16
NW = NC * NS

CPW = B // NW
CC = 8
NT = CPW // CC
PC = CC * NCTX
IW = 80
NIR = PC // IW
NQ = D // L


def _rsqrt(t):
    ti = plsc.bitcast(t, jnp.int32)
    y = plsc.bitcast(jnp.int32(0x5F3759DF) - (ti >> 1), jnp.float32)
    for _ in range(2):
        y = y * (1.5 - 0.5 * t * y * y)
    return y


def _sc_body(centers_hbm, contexts_hbm, cidx_hbm, ctxidx_hbm, negidx_hbm,
             out_hbm,
             cidx_v, ctxidx_v, negidx_v, c_rows, ctx_rows, neg_rows,
             n2c_buf, dp_buf, n2x_buf, dn_buf, n2n_buf, kidx_v, acc_v,
             *sems):
    w = lax.axis_index("s") * NC + lax.axis_index("c")
    last_lane = lax.iota(jnp.int32, L) == (L - 1)
    lane = lax.iota(jnp.int32, L)
    for g in range(PC // L):
        kidx_v[pl.ds(g * L, L)] = (g * L + lane) // NCTX

    def load_indices():
        cbase = pl.multiple_of(w * NT, NT)
        rbase = pl.multiple_of(w * (NT * NIR), NT * NIR)
        cs = [
            pltpu.async_copy(cidx_hbm.at[pl.ds(cbase, NT)], cidx_v, sems[0]),
            pltpu.async_copy(ctxidx_hbm.at[pl.ds(rbase, NT * NIR)], ctxidx_v,
                             sems[0]),
            pltpu.async_copy(negidx_hbm.at[pl.ds(rbase, NT * NIR)], negidx_v,
                             sems[0]),
        ]
        for c in cs:
            c.wait()

    def stage(t, s):
        cs = [pltpu.async_copy(
            centers_hbm.at[cidx_v.at[t]], c_rows.at[s], sems[s])]
        for j in range(NIR):
            cs.append(pltpu.async_copy(
                contexts_hbm.at[ctxidx_v.at[t * NIR + j]],
                ctx_rows.at[s].at[pl.ds(j * IW, IW)], sems[s]))
            cs.append(pltpu.async_copy(
                contexts_hbm.at[negidx_v.at[t * NIR + j]],
                neg_rows.at[s].at[pl.ds(j * IW, IW)], sems[s]))
        return cs

    def store_total(buf, p, vec):
        plsc.store_scatter(buf, [jnp.full((L,), p, jnp.int32)],
                           plsc.cumsum(vec), mask=last_lane)

    def compute(s, acc):
        @plsc.parallel_loop(0, CC)
        def center_body(k):
            cqs = [c_rows[s, k, pl.ds(q * L, L)] for q in range(NQ)]
            n2cv = cqs[0] * cqs[0]
            for q in range(1, NQ):
                n2cv = n2cv + cqs[q] * cqs[q]
            store_total(n2c_buf, k, n2cv)

            @plsc.parallel_loop(0, NCTX, unroll=2)
            def pair_body(j):
                p = k * NCTX + j
                xv = ctx_rows[s, p, pl.ds(0, L)]
                nv = neg_rows[s, p, pl.ds(0, L)]
                dpv = xv * cqs[0]
                n2xv = xv * xv
                dnv = nv * cqs[0]
                n2nv = nv * nv
                for q in range(1, NQ):
                    xv = ctx_rows[s, p, pl.ds(q * L, L)]
                    nv = neg_rows[s, p, pl.ds(q * L, L)]
                    dpv = dpv + xv * cqs[q]
                    n2xv = n2xv + xv * xv
                    dnv = dnv + nv * cqs[q]
                    n2nv = n2nv + nv * nv
                store_total(dp_buf, p, dpv)
                store_total(n2x_buf, p, n2xv)
                store_total(dn_buf, p, dnv)
                store_total(n2n_buf, p, n2nv)


        @plsc.parallel_loop(0, PC // L, unroll=2, carry=acc)
        def grp_body(g, a):
            off = g * L
            dpv = dp_buf[pl.ds(off, L)]
            n2xv = n2x_buf[pl.ds(off, L)]
            dnv = dn_buf[pl.ds(off, L)]
            n2nv = n2n_buf[pl.ds(off, L)]
            kidx = kidx_v[pl.ds(off, L)]
            n2cv = plsc.load_gather(n2c_buf, [kidx])
            tp = n2cv * n2xv
            denp = jnp.maximum(tp * _rsqrt(tp), 1e-8)
            cosp = dpv / denp
            tn = n2cv * n2nv
            denn = jnp.maximum(tn * _rsqrt(tn), 1e-8)
            cosn = dnv / denn
            return a + (1.0 - cosp) + jnp.maximum(cosn, 0.0)

        return grp_body

    acc = jnp.zeros((L,), jnp.float32)
    load_indices()
    descs = {0: stage(0, 0)}
    for t in range(NT):
        s = t % 2
        if t + 1 < NT:
            descs[1 - s] = stage(t + 1, 1 - s)
        for c in descs[s]:
            c.wait()
        acc = compute(s, acc)

    acc_v[...] = acc * (1.0 / NPAIR)
    pltpu.sync_copy(acc_v, out_hbm.at[w])


_sc_kernel = functools.partial(
    pl.kernel,
    out_type=jax.ShapeDtypeStruct((NW, L), jnp.float32),
    mesh=plsc.VectorSubcoreMesh(core_axis_name="c", subcore_axis_name="s"),
    compiler_params=pltpu.CompilerParams(needs_layout_passes=False),
    scratch_types=[
        pltpu.VMEM((NT, CC), jnp.int32),
        pltpu.VMEM((NT * NIR, IW), jnp.int32),
        pltpu.VMEM((NT * NIR, IW), jnp.int32),
        pltpu.VMEM((2, CC, D), jnp.float32),
        pltpu.VMEM((2, PC, D), jnp.float32),
        pltpu.VMEM((2, PC, D), jnp.float32),
        pltpu.VMEM((CC,), jnp.float32),
        pltpu.VMEM((PC,), jnp.float32),
        pltpu.VMEM((PC,), jnp.float32),
        pltpu.VMEM((PC,), jnp.float32),
        pltpu.VMEM((PC,), jnp.float32),
        pltpu.VMEM((PC,), jnp.int32),
        pltpu.VMEM((L,), jnp.float32),
        pltpu.SemaphoreType.DMA,
        pltpu.SemaphoreType.DMA,
    ],
)(_sc_body)


@jax.jit
def kernel(centers, contexts, center_idxs, context_idxs, neg_idxs):
    cidx = center_idxs.astype(jnp.int32).reshape(B // CC, CC)
    ctxi = context_idxs.astype(jnp.int32).reshape(NPAIR // IW, IW)
    negi = neg_idxs.astype(jnp.int32).reshape(NPAIR // IW, IW)
    out = _sc_kernel(centers, contexts, cidx, ctxi, negi)
    return jnp.sum(out)
